# baseline (speedup 1.0000x reference)
import jax
import jax.numpy as jnp
from jax import lax
from jax.experimental import pallas as pl
from jax.experimental.pallas import tpu as pltpu

_B, _S, _F = 16, 2048, 128
_RB = 256
_NSTEP = _S // _RB
_BB = _B // _NSTEP
_NS = 8
_NL = _S // _NS
_BIG = float(jnp.finfo(jnp.float32).max)


def _shift1(v):
    t = jnp.concatenate(
        [jnp.full((_NS, 1), _BIG, jnp.float32), v[:, :_NL - 1]], axis=1)
    last = v[:, _NL - 1:_NL]
    u = jnp.concatenate(
        [jnp.full((1, 1), _BIG, jnp.float32), last[:_NS - 1, :]], axis=0)
    lane0 = lax.broadcasted_iota(jnp.int32, (_NS, _NL), 1) == 0
    return jnp.where(lane0, u, t)


def _cumsum_tile(v):
    k = 1
    while k < _NL:
        v = v + jnp.concatenate(
            [jnp.zeros((_NS, k), jnp.float32), v[:, :_NL - k]], axis=1)
        k *= 2
    t = v[:, _NL - 1:_NL]
    off = jnp.concatenate(
        [jnp.zeros((1, 1), jnp.float32), t[:_NS - 1, :]], axis=0)
    k = 1
    while k < _NS:
        off = off + jnp.concatenate(
            [jnp.zeros((k, 1), jnp.float32), off[:_NS - k, :]], axis=0)
        k *= 2
    return v + off


def _cummin_tile(v):
    k = 1
    while k < _NL:
        v = jnp.minimum(v, jnp.concatenate(
            [jnp.full((_NS, k), _BIG, jnp.float32), v[:, :_NL - k]], axis=1))
        k *= 2
    t = v[:, _NL - 1:_NL]
    off = jnp.concatenate(
        [jnp.full((1, 1), _BIG, jnp.float32), t[:_NS - 1, :]], axis=0)
    k = 1
    while k < _NS:
        off = jnp.minimum(off, jnp.concatenate(
            [jnp.full((k, 1), _BIG, jnp.float32), off[:_NS - k, :]], axis=0))
        k *= 2
    return jnp.minimum(v, off)


def _row_update(prev, drow):
    shifted = _shift1(prev)
    m = jnp.minimum(prev, shifted)
    b = drow + m
    c = _cumsum_tile(drow)
    return c + _cummin_tile(b - c)


def _fused_kernel(pred_ref, target_ref, x_ref, y_ref, out_ref,
                  g_scr, dprev_scr, acc_ref):
    i = pl.program_id(0)

    part = jnp.sum(jnp.abs(pred_ref[...] - target_ref[...]))

    @pl.when(i == 0)
    def _():
        acc_ref[0] = 0.0

    acc_ref[0] = acc_ref[0] + part

    xb = x_ref[...]
    y = y_ref[...]
    xsq = jnp.sum(xb * xb, axis=1, keepdims=True)
    ysq = jnp.sum(y * y, axis=1, keepdims=True)
    lhs = jnp.concatenate(
        [-2.0 * xb, xsq, jnp.ones((_RB, 1), jnp.float32)], axis=1)
    rhs = jnp.concatenate(
        [y, jnp.ones((_S, 1), jnp.float32), ysq], axis=1)
    sq = lax.dot_general(lhs, rhs, (((1,), (1,)), ((), ())),
                         preferred_element_type=jnp.float32)
    d = jnp.sqrt(jnp.maximum(sq, 1e-12))
    for s in range(_NS):
        g_scr[s] = d[:, s * _NL:(s + 1) * _NL]

    def gather_rows(base):
        strips = [g_scr[s, pl.ds(base, 8), :] for s in range(_NS)]
        return [jnp.concatenate(
                    [strips[s][r8:r8 + 1, :] for s in range(_NS)], axis=0)
                for r8 in range(8)]

    @pl.when(i == 0)
    def _():
        rows = gather_rows(0)
        row = _cumsum_tile(rows[0])
        for r8 in range(1, 8):
            row = _row_update(row, rows[r8])
        dprev_scr[...] = row

    start = jnp.where(i == 0, 1, 0)

    def outer(rt, carry):
        base = pl.multiple_of(rt * 8, 8)
        rows = gather_rows(base)
        for r8 in range(8):
            carry = _row_update(carry, rows[r8])
        return carry

    final = lax.fori_loop(start, _RB // 8, outer, dprev_scr[...])
    dprev_scr[...] = final

    @pl.when(i == _NSTEP - 1)
    def _():
        mae = acc_ref[0] / float(_B * _S * _F)
        dtw = final[_NS - 1, _NL - 1] / float(_S * _F)
        out_ref[...] = (0.5 * mae + 0.5 * dtw) * jnp.ones((1, 1), jnp.float32)


def kernel(pred, target):
    x = pred[0]
    y = target[0]
    out = pl.pallas_call(
        _fused_kernel,
        grid=(_NSTEP,),
        in_specs=[
            pl.BlockSpec((_BB, _S, _F), lambda i: (i, 0, 0)),
            pl.BlockSpec((_BB, _S, _F), lambda i: (i, 0, 0)),
            pl.BlockSpec((_RB, _F), lambda i: (i, 0)),
            pl.BlockSpec((_S, _F), lambda i: (0, 0)),
        ],
        out_specs=pl.BlockSpec((1, 1), lambda i: (0, 0)),
        out_shape=jax.ShapeDtypeStruct((1, 1), jnp.float32),
        scratch_shapes=[
            pltpu.VMEM((_NS, _RB, _NL), jnp.float32),
            pltpu.VMEM((_NS, _NL), jnp.float32),
            pltpu.SMEM((1,), jnp.float32),
        ],
        compiler_params=pltpu.CompilerParams(
            dimension_semantics=("arbitrary",),
        ),
    )(pred, target, x, y)
    return out[0, 0]


# speedup vs baseline: 2.0023x; 2.0023x over previous
import functools

import jax
import jax.numpy as jnp
from jax import lax
from jax.experimental import pallas as pl
from jax.experimental.pallas import tpu as pltpu

_B, _S, _F = 16, 2048, 128
_RB = 256
_NSTEP = _S // _RB
_BB = _B // _NSTEP
_NS = 8
_NL = _S // _NS
_BIG = float(jnp.finfo(jnp.float32).max)


def _min_tree(vals):
    while len(vals) > 1:
        nxt = [jnp.minimum(vals[k], vals[k + 1])
               for k in range(0, len(vals) - 1, 2)]
        if len(vals) % 2:
            nxt.append(vals[-1])
        vals = nxt
    return vals[0]


def _cumsum_cm(v):
    for k in (1, 2, 4):
        v = v + jnp.concatenate(
            [jnp.zeros((k, _NL), jnp.float32), v[:_NS - k, :]], axis=0)
    t = v[_NS - 1:_NS, :]
    te = jnp.concatenate(
        [jnp.zeros((1, 1), jnp.float32), t[:, :_NL - 1]], axis=1)
    k = 1
    while k < _NL:
        te = te + jnp.concatenate(
            [jnp.zeros((1, k), jnp.float32), te[:, :_NL - k]], axis=1)
        k *= 2
    return v + te


def _cummin_cm(v):
    for k in (1, 2, 4):
        v = jnp.minimum(v, jnp.concatenate(
            [jnp.full((k, _NL), _BIG, jnp.float32), v[:_NS - k, :]], axis=0))
    t = v[_NS - 1:_NS, :]
    te = jnp.concatenate(
        [jnp.full((1, 1), _BIG, jnp.float32), t[:, :_NL - 1]], axis=1)
    for step in (1, 16):
        shifts = [jnp.concatenate(
                      [jnp.full((1, step * m), _BIG, jnp.float32),
                       te[:, :_NL - step * m]], axis=1)
                  for m in range(1, 16)]
        te = _min_tree([te] + shifts)
    return jnp.minimum(v, te)


def _row_update(prev, drow, c):
    m1 = jnp.minimum(prev, jnp.concatenate(
        [jnp.full((1, _NL), _BIG, jnp.float32), prev[:_NS - 1, :]], axis=0))
    r7 = jnp.concatenate(
        [jnp.full((1, 1), _BIG, jnp.float32), prev[_NS - 1:_NS, :_NL - 1]],
        axis=1)
    sub0 = lax.broadcasted_iota(jnp.int32, (_NS, _NL), 0) == 0
    m = jnp.where(sub0, jnp.minimum(m1, r7), m1)
    b = drow + m
    return c + _cummin_cm(b - c)


def _fused_kernel(pred_ref, target_ref, x_ref, y_ref, out_ref,
                  g_scr, dprev_scr, acc_ref):
    i = pl.program_id(0)

    part = jnp.sum(jnp.abs(pred_ref[...] - target_ref[...]))

    @pl.when(i == 0)
    def _():
        acc_ref[0] = 0.0

    acc_ref[0] = acc_ref[0] + part

    xb = x_ref[...]
    y = y_ref[...]
    xsq = jnp.sum(xb * xb, axis=1, keepdims=True)
    ysq = jnp.sum(y * y, axis=1, keepdims=True)
    lhs = jnp.concatenate(
        [-2.0 * xb, xsq, jnp.ones((_RB, 1), jnp.float32)], axis=1)
    rhs = jnp.concatenate(
        [y, jnp.ones((_S, 1), jnp.float32), ysq], axis=1)
    sq = lax.dot_general(lhs, rhs, (((1,), (1,)), ((), ())),
                         preferred_element_type=jnp.float32)
    d = jnp.sqrt(jnp.maximum(sq, 1e-12))
    for s in range(_NS):
        g_scr[s] = d[:, s * _NL:(s + 1) * _NL]

    def gather_rows(base):
        strips = [g_scr[s, pl.ds(base, 8), :] for s in range(_NS)]
        return [jnp.concatenate(
                    [strips[s][r8:r8 + 1, :] for s in range(_NS)], axis=0)
                for r8 in range(8)]

    @pl.when(i == 0)
    def _():
        rows = gather_rows(0)
        cs = [_cumsum_cm(r) for r in rows]
        row = cs[0]
        for r8 in range(1, 8):
            row = _row_update(row, rows[r8], cs[r8])
        dprev_scr[...] = row

    start = jnp.where(i == 0, 1, 0)

    def outer(rt, carry):
        base = pl.multiple_of(rt * 8, 8)
        rows = gather_rows(base)
        cs = [_cumsum_cm(r) for r in rows]
        for r8 in range(8):
            carry = _row_update(carry, rows[r8], cs[r8])
        return carry

    final = lax.fori_loop(start, _RB // 8, outer, dprev_scr[...])
    dprev_scr[...] = final

    @pl.when(i == _NSTEP - 1)
    def _():
        mae = acc_ref[0] / float(_B * _S * _F)
        dtw = final[_NS - 1, _NL - 1] / float(_S * _F)
        out_ref[...] = (0.5 * mae + 0.5 * dtw) * jnp.ones((1, 1), jnp.float32)


def kernel(pred, target):
    x = pred[0]
    perm = (jnp.arange(_S, dtype=jnp.int32) % _NL) * _NS \
        + jnp.arange(_S, dtype=jnp.int32) // _NL
    y = target[0][perm]
    out = pl.pallas_call(
        _fused_kernel,
        grid=(_NSTEP,),
        in_specs=[
            pl.BlockSpec((_BB, _S, _F), lambda i: (i, 0, 0)),
            pl.BlockSpec((_BB, _S, _F), lambda i: (i, 0, 0)),
            pl.BlockSpec((_RB, _F), lambda i: (i, 0)),
            pl.BlockSpec((_S, _F), lambda i: (0, 0)),
        ],
        out_specs=pl.BlockSpec((1, 1), lambda i: (0, 0)),
        out_shape=jax.ShapeDtypeStruct((1, 1), jnp.float32),
        scratch_shapes=[
            pltpu.VMEM((_NS, _RB, _NL), jnp.float32),
            pltpu.VMEM((_NS, _NL), jnp.float32),
            pltpu.SMEM((1,), jnp.float32),
        ],
        compiler_params=pltpu.CompilerParams(
            dimension_semantics=("arbitrary",),
        ),
    )(pred, target, x, y)
    return out[0, 0]


# speedup vs baseline: 2.5223x; 1.2597x over previous
import jax
import jax.numpy as jnp
from jax import lax
from jax.experimental import pallas as pl
from jax.experimental.pallas import tpu as pltpu

_B, _S, _F = 16, 2048, 128
_RB = 256
_NSTEP = _S // _RB
_BB = _B // _NSTEP
_NS = 8
_NL = _S // _NS
_BIG = float(jnp.finfo(jnp.float32).max)


def _min_tree(vals):
    while len(vals) > 1:
        nxt = [jnp.minimum(vals[k], vals[k + 1])
               for k in range(0, len(vals) - 1, 2)]
        if len(vals) % 2:
            nxt.append(vals[-1])
        vals = nxt
    return vals[0]


def _dotf(a, b, dims):
    return lax.dot_general(a, b, (dims, ((), ())),
                           preferred_element_type=jnp.float32)


def _cummin_cm(v):
    for k in (1, 2, 4):
        v = jnp.minimum(v, jnp.concatenate(
            [jnp.full((k, _NL), _BIG, jnp.float32), v[:_NS - k, :]], axis=0))
    t = v[_NS - 1:_NS, :]
    r1 = [jnp.concatenate(
              [jnp.full((1, m), _BIG, jnp.float32), t[:, :_NL - m]], axis=1)
          for m in range(1, 17)]
    te = _min_tree(r1)
    r2 = [te] + [jnp.concatenate(
                     [jnp.full((1, 16 * m), _BIG, jnp.float32),
                      te[:, :_NL - 16 * m]], axis=1)
                 for m in range(1, 16)]
    te = _min_tree(r2)
    return jnp.minimum(v, te)


def _row_update(prev, drow, c):
    m1 = jnp.minimum(prev, jnp.concatenate(
        [jnp.full((1, _NL), _BIG, jnp.float32), prev[:_NS - 1, :]], axis=0))
    r7 = jnp.concatenate(
        [jnp.full((1, 1), _BIG, jnp.float32), prev[_NS - 1:_NS, :_NL - 1]],
        axis=1)
    sub0 = lax.broadcasted_iota(jnp.int32, (_NS, _NL), 0) == 0
    m = jnp.where(sub0, jnp.minimum(m1, r7), m1)
    b = drow + m
    return c + _cummin_cm(b - c)


def _fused_kernel(pred_ref, target_ref, x_ref, y_ref,
                  perm_ref, lp_ref, op_ref, o8_ref, uex_ref, out_ref,
                  g_scr, dprev_scr, acc_ref):
    i = pl.program_id(0)

    part = jnp.sum(jnp.abs(pred_ref[...] - target_ref[...]))

    @pl.when(i == 0)
    def _():
        acc_ref[0] = 0.0

    acc_ref[0] = acc_ref[0] + part

    xb = x_ref[...]
    y = y_ref[...]
    xsq = jnp.sum(xb * xb, axis=1, keepdims=True)
    ysq = jnp.sum(y * y, axis=1, keepdims=True)
    lhs = jnp.concatenate(
        [-2.0 * xb, xsq, jnp.ones((_RB, 1), jnp.float32)], axis=1)
    rhs = jnp.concatenate(
        [y, jnp.ones((_S, 1), jnp.float32), ysq], axis=1)
    sq = lax.dot_general(lhs, rhs, (((1,), (1,)), ((), ())),
                         preferred_element_type=jnp.float32)
    d = jnp.sqrt(jnp.maximum(sq, 1e-12))
    for s in range(_NS):
        g_scr[s] = d[:, s * _NL:(s + 1) * _NL]

    perm = perm_ref[...]
    lp = lp_ref[...]
    op = op_ref[...]
    o8 = o8_ref[...]
    uex = uex_ref[...]

    def load_group(base):
        s_cat = g_scr[:, pl.ds(base, 8), :].reshape(_NS * 8, _NL)
        v_cat = _dotf(perm, s_cat, ((1,), (0,)))
        incol = _dotf(lp, s_cat, ((1,), (0,)))
        tall = _dotf(op, s_cat, ((1,), (0,)))
        te_all = _dotf(tall, uex, ((1,), (0,)))
        tebc = _dotf(o8, te_all, ((0,), (0,)))
        c_cat = incol + tebc
        rows = [v_cat[8 * r8:8 * r8 + 8, :] for r8 in range(8)]
        cs = [c_cat[8 * r8:8 * r8 + 8, :] for r8 in range(8)]
        return rows, cs

    @pl.when(i == 0)
    def _():
        rows, cs = load_group(0)
        row = cs[0]
        for r8 in range(1, 8):
            row = _row_update(row, rows[r8], cs[r8])
        dprev_scr[...] = row

    start = jnp.where(i == 0, 1, 0)

    def outer(rt, carry):
        base = pl.multiple_of(rt * 8, 8)
        rows, cs = load_group(base)
        for r8 in range(8):
            carry = _row_update(carry, rows[r8], cs[r8])
        return carry

    final = lax.fori_loop(start, _RB // 8, outer, dprev_scr[...])
    dprev_scr[...] = final

    @pl.when(i == _NSTEP - 1)
    def _():
        mae = acc_ref[0] / float(_B * _S * _F)
        dtw = final[_NS - 1, _NL - 1] / float(_S * _F)
        out_ref[...] = (0.5 * mae + 0.5 * dtw) * jnp.ones((1, 1), jnp.float32)


def kernel(pred, target):
    x = pred[0]
    pj = (jnp.arange(_S, dtype=jnp.int32) % _NL) * _NS \
        + jnp.arange(_S, dtype=jnp.int32) // _NL
    y = target[0][pj]

    k = jnp.arange(64)
    ksrc = (k % 8) * 8 + k // 8
    perm = (ksrc[:, None] == jnp.arange(64)[None, :]).astype(jnp.float32)
    lbd = ((k[:, None] // 8 == k[None, :] // 8)
           & (k[None, :] % 8 <= k[:, None] % 8)).astype(jnp.float32)
    lp = lbd @ perm
    o8 = (jnp.arange(8)[:, None] == k[None, :] // 8).astype(jnp.float32)
    op = o8 @ perm
    cl = jnp.arange(_NL)
    uex = (cl[:, None] < cl[None, :]).astype(jnp.float32)

    out = pl.pallas_call(
        _fused_kernel,
        grid=(_NSTEP,),
        in_specs=[
            pl.BlockSpec((_BB, _S, _F), lambda i: (i, 0, 0)),
            pl.BlockSpec((_BB, _S, _F), lambda i: (i, 0, 0)),
            pl.BlockSpec((_RB, _F), lambda i: (i, 0)),
            pl.BlockSpec((_S, _F), lambda i: (0, 0)),
            pl.BlockSpec((64, 64), lambda i: (0, 0)),
            pl.BlockSpec((64, 64), lambda i: (0, 0)),
            pl.BlockSpec((8, 64), lambda i: (0, 0)),
            pl.BlockSpec((8, 64), lambda i: (0, 0)),
            pl.BlockSpec((_NL, _NL), lambda i: (0, 0)),
        ],
        out_specs=pl.BlockSpec((1, 1), lambda i: (0, 0)),
        out_shape=jax.ShapeDtypeStruct((1, 1), jnp.float32),
        scratch_shapes=[
            pltpu.VMEM((_NS, _RB, _NL), jnp.float32),
            pltpu.VMEM((_NS, _NL), jnp.float32),
            pltpu.SMEM((1,), jnp.float32),
        ],
        compiler_params=pltpu.CompilerParams(
            dimension_semantics=("arbitrary",),
        ),
    )(pred, target, x, y, perm, lp, op, o8, uex)
    return out[0, 0]
